# baseline (speedup 1.0000x reference)
import functools

import jax
import jax.numpy as jnp
from jax import lax
from jax.experimental import pallas as pl
from jax.experimental.pallas import tpu as pltpu
from jax.experimental.pallas import tpu_sc as plsc

N = 100000
D = 128
NW = 32
WPW = 3136
CH = 64
NG = CH // 16

R0, R1, R2, R3 = 119, 5 * 12, 12 * 10, 6 * 6 * 2 * 2


def _build_tables_body(e0, e1, e2, e3, e4, e5, e6, e7, e8, o0, o1, o2, o3):
    o0[...] = e0[...]
    o1[...] = (e1[...][:, None, :] + e2[...][None, :, :]).reshape(R1, D)
    o2[...] = (e3[...][:, None, :] + e4[...][None, :, :]).reshape(R2, D)
    t56 = (e5[...][:, None, :] + e6[...][None, :, :]).reshape(36, D)
    t78 = (e7[...][:, None, :] + e8[...][None, :, :]).reshape(4, D)
    o3[...] = (t56[:, None, :] + t78[None, :, :]).reshape(R3, D)


_build_tables = pl.pallas_call(
    _build_tables_body,
    out_shape=(
        jax.ShapeDtypeStruct((R0, D), jnp.float32),
        jax.ShapeDtypeStruct((R1, D), jnp.float32),
        jax.ShapeDtypeStruct((R2, D), jnp.float32),
        jax.ShapeDtypeStruct((R3, D), jnp.float32),
    ),
)


def _sc_main(xf, t0, t1, t2, t3):
    mesh = plsc.VectorSubcoreMesh(core_axis_name="c", subcore_axis_name="s")

    @functools.partial(
        pl.kernel,
        mesh=mesh,
        out_type=jax.ShapeDtypeStruct((N, D), jnp.float32),
        compiler_params=pltpu.CompilerParams(needs_layout_passes=False),
        scratch_types=[
            [pltpu.VMEM((CH * 9,), jnp.int32) for _ in range(2)],
            [[pltpu.VMEM((CH,), jnp.int32) for _ in range(4)]
             for _ in range(2)],
            [[pltpu.VMEM((CH, D), jnp.float32) for _ in range(3)]
             for _ in range(2)],
            [pltpu.VMEM((CH, D), jnp.float32) for _ in range(2)],
            [pltpu.SemaphoreType.DMA for _ in range(2)],
            [pltpu.SemaphoreType.DMA for _ in range(2)],
        ],
    )
    def k(x_hbm, t0_hbm, t1_hbm, t2_hbm, t3_hbm, out_hbm,
          xs, idx, rb, os, gsem, osem):
        wid = lax.axis_index("s") * 2 + lax.axis_index("c")
        base_w = wid * WPW
        rows_w = jnp.minimum(N - base_w, WPW)
        nch = (rows_w + (CH - 1)) // CH

        iot = lax.iota(jnp.int32, 16)
        tbs = (t0_hbm, t1_hbm, t2_hbm, t3_hbm)

        def chunk_base(ci):
            return jnp.minimum(base_w + ci * CH, N - CH)

        def fire(ci, s):
            base = chunk_base(ci)
            pltpu.sync_copy(x_hbm.at[pl.ds(base * 9, CH * 9)], xs[s])
            for g in range(NG):
                ax = (iot + g * 16) * 9
                xv = [plsc.load_gather(xs[s], [ax + j]) for j in range(9)]
                a = [
                    xv[0],
                    xv[1] * 12 + xv[2],
                    xv[3] * 10 + xv[4],
                    xv[5] * 24 + xv[6] * 4 + xv[7] * 2 + xv[8],
                ]
                for t in range(4):
                    idx[s][t][pl.ds(g * 16, 16)] = a[t]
            pltpu.async_copy(tbs[0].at[idx[s][0]], os[s], gsem[s])
            for t in range(1, 4):
                pltpu.async_copy(tbs[t].at[idx[s][t]], rb[s][t - 1], gsem[s])

        def accum(ci, s):
            base = chunk_base(ci)
            pltpu.make_async_copy(tbs[0].at[idx[s][0]], os[s], gsem[s]).wait()
            for t in range(1, 4):
                pltpu.make_async_copy(
                    tbs[t].at[idx[s][t]], rb[s][t - 1], gsem[s]).wait()

            @plsc.parallel_loop(0, NG)
            def group_body(g):
                for i_ in range(16):
                    i = g * 16 + i_
                    for c0 in range(0, D, 16):
                        v = os[s][i, pl.ds(c0, 16)]
                        v = v + rb[s][0][i, pl.ds(c0, 16)]
                        v = v + rb[s][1][i, pl.ds(c0, 16)]
                        v = v + rb[s][2][i, pl.ds(c0, 16)]
                        os[s][i, pl.ds(c0, 16)] = v

            pltpu.async_copy(os[s], out_hbm.at[pl.ds(base, CH)], osem[s])

        def wait_out(s):
            pltpu.make_async_copy(
                os[s], out_hbm.at[pl.ds(0, CH)], osem[s]).wait()

        fire(0, 0)

        def pair_body(p, carry):
            ci = p * 2

            @pl.when((ci + 1 < nch) & (p > 0))
            def _():
                wait_out(1)

            @pl.when(ci + 1 < nch)
            def _():
                fire(ci + 1, 1)

            accum(ci, 0)

            @pl.when(ci + 2 < nch)
            def _():
                wait_out(0)
                fire(ci + 2, 0)

            @pl.when(ci + 1 < nch)
            def _():
                accum(ci + 1, 1)

            return carry

        lax.fori_loop(0, (nch + 1) // 2, pair_body, 0)
        wait_out(0)

        @pl.when(nch > 1)
        def _():
            wait_out(1)

    return k(xf, t0, t1, t2, t3)


def kernel(x, emb0, emb1, emb2, emb3, emb4, emb5, emb6, emb7, emb8):
    t0, t1, t2, t3 = _build_tables(emb0, emb1, emb2, emb3, emb4,
                                   emb5, emb6, emb7, emb8)
    return _sc_main(x.reshape(N * 9), t0, t1, t2, t3)
